# baseline (speedup 1.0000x reference)
import functools

import jax
import jax.numpy as jnp
from jax import lax
from jax.experimental import pallas as pl
from jax.experimental.pallas import tpu as pltpu
from jax.experimental.pallas import tpu_sc as plsc

B = 4
S = 4096
D = 1024
PAD = 1
NC = 2
NS = 16
L = 16
NW = NC * NS
JW = S // NW
C = 16
NCH = JW // C

_mesh = plsc.VectorSubcoreMesh(core_axis_name="c", subcore_axis_name="s")


@functools.partial(
    pl.kernel,
    out_type=jax.ShapeDtypeStruct((B, S, D), jnp.float32),
    mesh=_mesh,
    compiler_params=pltpu.CompilerParams(needs_layout_passes=False),
    scratch_types=[
        pltpu.VMEM((B, JW), jnp.int32),
        pltpu.VMEM((JW,), jnp.int32),
        pltpu.VMEM((2, C, D), jnp.float32),
        pltpu.VMEM((8, D), jnp.float32),
        pltpu.VMEM((L, D), jnp.float32),
        pltpu.SemaphoreType.DMA,
        pltpu.SemaphoreType.DMA,
        pltpu.SemaphoreType.DMA,
        pltpu.SemaphoreType.DMA,
    ],
)
def _sinus_embed(x_hbm, w_hbm, idx_hbm, out_hbm, xbuf, idxvm, wbuf, padbuf,
                 tbuf, rsem, wsem0, wsem1, ssem):
    wid = lax.axis_index("s") * NC + lax.axis_index("c")
    j0 = wid * JW
    wsems = (wsem0, wsem1)

    idx_desc = pltpu.async_copy(idx_hbm.at[pl.ds(j0, JW)], idxvm, rsem)
    x_desc = pltpu.async_copy(x_hbm.at[:, pl.ds(j0, JW)], xbuf, ssem)
    pad_desc = pltpu.async_copy(w_hbm.at[pl.ds(0, 8)], padbuf, ssem)
    idx_desc.wait()

    read_descs = [None] * NCH
    write_descs = [None] * NCH
    read_descs[0] = pltpu.async_copy(
        w_hbm.at[idxvm.at[pl.ds(0, C)]], wbuf.at[0], rsem
    )

    for c in range(NCH):
        buf = c % 2
        read_descs[c].wait()
        if c + 1 < NCH:
            if c >= 1:
                for d in write_descs[c - 1]:
                    d.wait()
                write_descs[c - 1] = None
            read_descs[c + 1] = pltpu.async_copy(
                w_hbm.at[idxvm.at[pl.ds((c + 1) * C, C)]], wbuf.at[1 - buf], rsem
            )
        jc = j0 + c * C
        write_descs[c] = [
            pltpu.async_copy(
                wbuf.at[buf], out_hbm.at[b, pl.ds(jc, C)], wsems[buf]
            )
            for b in range(B)
        ]

    for descs in write_descs:
        if descs is not None:
            for d in descs:
                d.wait()
    x_desc.wait()
    pad_desc.wait()

    lane = lax.broadcasted_iota(jnp.int32, (L,), 0)
    gpb = JW // L

    for b in range(B):

        def group_body(g, _, b=b):
            jg = j0 + g * L
            xv = xbuf[b, pl.ds(g * L, L)]
            padv = jnp.where(xv == PAD, 1, 0).astype(jnp.int32)
            npad = jnp.sum(padv)

            @pl.when(npad != 0)
            def _fix():
                pltpu.sync_copy(out_hbm.at[b, pl.ds(jg, L)], tbuf)
                for r in range(L):
                    is_pad = jnp.sum(jnp.where(lane == r, padv, 0))

                    @pl.when(is_pad != 0)
                    def _patch(r=r):
                        def d_body(dd, _):
                            tbuf[r, pl.ds(dd * L, L)] = padbuf[PAD, pl.ds(dd * L, L)]
                            return 0

                        lax.fori_loop(0, D // L, d_body, 0)

                pltpu.sync_copy(tbuf, out_hbm.at[b, pl.ds(jg, L)])

            return 0

        lax.fori_loop(0, gpb, group_body, 0)


def kernel(X, weights):
    idx = jnp.arange(2, S + 2, dtype=jnp.int32)
    return _sinus_embed(X, weights, idx)


# speedup vs baseline: 1.0510x; 1.0510x over previous
import functools

import jax
import jax.numpy as jnp
from jax import lax
from jax.experimental import pallas as pl
from jax.experimental.pallas import tpu as pltpu
from jax.experimental.pallas import tpu_sc as plsc

B = 4
S = 4096
D = 1024
PAD = 1
NC = 2
NS = 16
L = 16
NW = NC * NS
JW = S // NW
C = 32
NCH = JW // C

_mesh = plsc.VectorSubcoreMesh(core_axis_name="c", subcore_axis_name="s")


@functools.partial(
    pl.kernel,
    out_type=jax.ShapeDtypeStruct((B, S, D), jnp.float32),
    mesh=_mesh,
    compiler_params=pltpu.CompilerParams(needs_layout_passes=False),
    scratch_types=[
        pltpu.VMEM((B, JW), jnp.int32),
        pltpu.VMEM((JW,), jnp.int32),
        pltpu.VMEM((3, C, D), jnp.float32),
        pltpu.VMEM((8, D), jnp.float32),
        pltpu.VMEM((L, D), jnp.float32),
        pltpu.SemaphoreType.DMA,
        pltpu.SemaphoreType.DMA,
        pltpu.SemaphoreType.DMA,
        pltpu.SemaphoreType.DMA,
        pltpu.SemaphoreType.DMA,
    ],
)
def _sinus_embed(x_hbm, w_hbm, idx_hbm, out_hbm, xbuf, idxvm, wbuf, padbuf,
                 tbuf, rsem, wsem0, wsem1, wsem2, ssem):
    wid = lax.axis_index("s") * NC + lax.axis_index("c")
    j0 = wid * JW
    wsems = (wsem0, wsem1, wsem2)

    idx_desc = pltpu.async_copy(idx_hbm.at[pl.ds(j0, JW)], idxvm, rsem)
    x_desc = pltpu.async_copy(x_hbm.at[:, pl.ds(j0, JW)], xbuf, ssem)
    pad_desc = pltpu.async_copy(w_hbm.at[pl.ds(0, 8)], padbuf, ssem)
    idx_desc.wait()

    read_descs = [None] * NCH
    write_descs = [None] * NCH
    read_descs[0] = pltpu.async_copy(
        w_hbm.at[idxvm.at[pl.ds(0, C)]], wbuf.at[0], rsem
    )

    for c in range(NCH):
        buf = c % 3
        read_descs[c].wait()
        if c + 1 < NCH:
            if c >= 2:
                for d in write_descs[c - 2]:
                    d.wait()
                write_descs[c - 2] = None
            read_descs[c + 1] = pltpu.async_copy(
                w_hbm.at[idxvm.at[pl.ds((c + 1) * C, C)]], wbuf.at[(c + 1) % 3], rsem
            )
        jc = j0 + c * C
        write_descs[c] = [
            pltpu.async_copy(
                wbuf.at[buf], out_hbm.at[b, pl.ds(jc, C)], wsems[buf]
            )
            for b in range(B)
        ]

    for descs in write_descs:
        if descs is not None:
            for d in descs:
                d.wait()
    x_desc.wait()
    pad_desc.wait()

    lane = lax.broadcasted_iota(jnp.int32, (L,), 0)
    gpb = JW // L

    for b in range(B):

        def group_body(g, _, b=b):
            jg = j0 + g * L
            xv = xbuf[b, pl.ds(g * L, L)]
            padv = jnp.where(xv == PAD, 1, 0).astype(jnp.int32)
            npad = jnp.sum(padv)

            @pl.when(npad != 0)
            def _fix():
                pltpu.sync_copy(out_hbm.at[b, pl.ds(jg, L)], tbuf)
                for r in range(L):
                    is_pad = jnp.sum(jnp.where(lane == r, padv, 0))

                    @pl.when(is_pad != 0)
                    def _patch(r=r):
                        def d_body(dd, _):
                            tbuf[r, pl.ds(dd * L, L)] = padbuf[PAD, pl.ds(dd * L, L)]
                            return 0

                        lax.fori_loop(0, D // L, d_body, 0)

                pltpu.sync_copy(tbuf, out_hbm.at[b, pl.ds(jg, L)])

            return 0

        lax.fori_loop(0, gpb, group_body, 0)


def kernel(X, weights):
    idx = jnp.arange(2, S + 2, dtype=jnp.int32)
    return _sinus_embed(X, weights, idx)


# speedup vs baseline: 1.1270x; 1.0723x over previous
import functools

import jax
import jax.numpy as jnp
from jax import lax
from jax.experimental import pallas as pl
from jax.experimental.pallas import tpu as pltpu
from jax.experimental.pallas import tpu_sc as plsc

B = 4
S = 4096
D = 1024
PAD = 1
NC = 2
NS = 16
L = 16
NW = NC * NS
JW = S // NW
C = 32
NCH = JW // C

_mesh = plsc.VectorSubcoreMesh(core_axis_name="c", subcore_axis_name="s")


@functools.partial(
    pl.kernel,
    out_type=jax.ShapeDtypeStruct((B, S, D), jnp.float32),
    mesh=_mesh,
    compiler_params=pltpu.CompilerParams(needs_layout_passes=False),
    scratch_types=[
        pltpu.VMEM((B, JW), jnp.int32),
        pltpu.VMEM((JW,), jnp.int32),
        pltpu.VMEM((3, C, D), jnp.float32),
        pltpu.VMEM((8, D), jnp.float32),
        pltpu.VMEM((L, D), jnp.float32),
        pltpu.SemaphoreType.DMA,
        pltpu.SemaphoreType.DMA,
        pltpu.SemaphoreType.DMA,
        pltpu.SemaphoreType.DMA,
        pltpu.SemaphoreType.DMA,
    ],
)
def _sinus_embed(x_hbm, w_hbm, idx_hbm, out_hbm, xbuf, idxvm, wbuf, padbuf,
                 tbuf, rsem, wsem0, wsem1, wsem2, ssem):
    wid = lax.axis_index("s") * NC + lax.axis_index("c")
    j0 = wid * JW
    wsems = (wsem0, wsem1, wsem2)

    idx_desc = pltpu.async_copy(idx_hbm.at[pl.ds(j0, JW)], idxvm, rsem)
    x_desc = pltpu.async_copy(x_hbm.at[:, pl.ds(j0, JW)], xbuf, ssem)
    pad_desc = pltpu.async_copy(w_hbm.at[pl.ds(0, 8)], padbuf, ssem)
    idx_desc.wait()

    lane = lax.broadcasted_iota(jnp.int32, (L,), 0)
    gpb = JW // L

    read_descs = [None] * NCH
    write_descs = [None] * NCH
    read_descs[0] = pltpu.async_copy(
        w_hbm.at[idxvm.at[pl.ds(0, C)]], wbuf.at[0], rsem
    )

    for c in range(NCH):
        buf = c % 3
        read_descs[c].wait()
        if c + 1 < NCH:
            if c >= 2:
                for d in write_descs[c - 2]:
                    d.wait()
                write_descs[c - 2] = None
            read_descs[c + 1] = pltpu.async_copy(
                w_hbm.at[idxvm.at[pl.ds((c + 1) * C, C)]], wbuf.at[(c + 1) % 3], rsem
            )
        jc = j0 + c * C
        write_descs[c] = [
            pltpu.async_copy(
                wbuf.at[buf], out_hbm.at[b, pl.ds(jc, C)], wsems[buf]
            )
            for b in range(B)
        ]
        if c == 0:
            x_desc.wait()

            def scan_body(i, acc):
                b = i // gpb
                xv = xbuf[b, pl.ds((i - b * gpb) * L, L)]
                return acc + jnp.where(xv == PAD, 1, 0).astype(jnp.int32)

            hpad = jnp.sum(
                lax.fori_loop(0, B * gpb, scan_body,
                              jnp.zeros((L,), jnp.int32))
            )

    for descs in write_descs:
        if descs is not None:
            for d in descs:
                d.wait()
    pad_desc.wait()

    @pl.when(hpad != 0)
    def _fixup():
        def group_body(i, _):
            b = i // gpb
            g = i - b * gpb
            jg = j0 + g * L
            xv = xbuf[b, pl.ds(g * L, L)]
            padv = jnp.where(xv == PAD, 1, 0).astype(jnp.int32)
            npad = jnp.sum(padv)

            @pl.when(npad != 0)
            def _fix():
                pltpu.sync_copy(out_hbm.at[b, pl.ds(jg, L)], tbuf)

                def row_body(r, _):
                    is_pad = jnp.sum(jnp.where(lane == r, padv, 0))

                    @pl.when(is_pad != 0)
                    def _patch():
                        def d_body(dd, _):
                            tbuf[r, pl.ds(dd * L, L)] = padbuf[PAD, pl.ds(dd * L, L)]
                            return 0

                        lax.fori_loop(0, D // L, d_body, 0)

                    return 0

                lax.fori_loop(0, L, row_body, 0)
                pltpu.sync_copy(tbuf, out_hbm.at[b, pl.ds(jg, L)])

            return 0

        lax.fori_loop(0, B * gpb, group_body, 0)


def kernel(X, weights):
    idx = jnp.arange(2, S + 2, dtype=jnp.int32)
    return _sinus_embed(X, weights, idx)
